# baseline (speedup 1.0000x reference)
import functools

import jax
import jax.numpy as jnp
from jax import lax
from jax.experimental import pallas as pl
from jax.experimental.pallas import tpu as pltpu
from jax.experimental.pallas import tpu_sc as plsc

_N = 100000
_M = 20000
_C = 1000
_OUT = 16
_SEGW = 32
_ND = 16
_ACC = (_C + _ND) * _SEGW
_CH = 256
_NCHUNK = 480
_TOT = _NCHUNK * _CH
_PER_TILE = _NCHUNK // 32
_GPC = _CH // 128


def _mlp_body(x_ref, dx8_ref, dx12_ref, wd_ref, bd_ref, wp_ref, bp_ref,
              wf_ref, bf_ref, out_ref):
    x = x_ref[...]
    dem = jnp.maximum(
        jnp.dot(dx8_ref[...], wd_ref[...], preferred_element_type=jnp.float32)
        + bd_ref[...], 0.0)
    pur = jnp.maximum(
        jnp.dot(dx12_ref[...], wp_ref[...], preferred_element_type=jnp.float32)
        + bp_ref[...], 0.0)
    y60 = jnp.concatenate([dem, pur, x], axis=1)
    y = jnp.maximum(
        jnp.dot(y60, wf_ref[...], preferred_element_type=jnp.float32)
        + bf_ref[...], 0.0)
    b = y.shape[0]
    yp = jnp.concatenate(
        [y, jnp.ones((b, 1), jnp.float32), jnp.zeros((b, 11), jnp.float32)],
        axis=1)
    out_ref[...] = yp


def _run_mlp(x, dataset_x, w_dem, b_dem, w_pur, b_pur, w_feat, b_feat):
    blk = 4000
    grid = _N // blk
    full = lambda shape: pl.BlockSpec(shape, lambda i: (0, 0))
    return pl.pallas_call(
        _mlp_body,
        grid=(grid,),
        in_specs=[
            pl.BlockSpec((blk, 20), lambda i: (i, 0)),
            pl.BlockSpec((blk, 8), lambda i: (i, 0)),
            pl.BlockSpec((blk, 12), lambda i: (i, 0)),
            full((8, 20)), full((1, 20)),
            full((12, 20)), full((1, 20)),
            full((60, 20)), full((1, 20)),
        ],
        out_specs=pl.BlockSpec((blk, _SEGW), lambda i: (i, 0)),
        out_shape=jax.ShapeDtypeStruct((_N, _SEGW), jnp.float32),
    )(x, dataset_x[:, :8], dataset_x[:, 8:], w_dem, b_dem.reshape(1, 20),
      w_pur, b_pur.reshape(1, 20), w_feat, b_feat.reshape(1, 20))


def _seg_body(y_hbm, idx_hbm, seg_hbm, sum_out, max_out,
              acc_sum, acc_max_a, acc_max_b, idx_v, seg_v, rows_a, rows_b,
              sem_a, sem_b):
    cid = lax.axis_index("c")
    sid = lax.axis_index("s")
    wid = cid * 16 + sid
    zero16 = jnp.zeros((16,), jnp.float32)

    idx_cp = pltpu.async_copy(idx_hbm.at[pl.ds(wid * _GPC * _PER_TILE,
                                               _GPC * _PER_TILE)], idx_v, sem_a)
    seg_cp = pltpu.async_copy(seg_hbm.at[pl.ds(wid * _CH * _PER_TILE,
                                               _CH * _PER_TILE)], seg_v, sem_b)

    def init_body(i, carry):
        acc_sum[pl.ds(i * 16, 16)] = zero16
        acc_max_a[pl.ds(i * 16, 16)] = zero16
        acc_max_b[pl.ds(i * 16, 16)] = zero16
        return carry

    lax.fori_loop(0, _ACC // 16, init_body, 0)
    idx_cp.wait()
    seg_cp.wait()

    bufs = (rows_a, rows_b)
    sems = (sem_a, sem_b)
    maxs = (acc_max_a, acc_max_b)

    def fire(j, buf, sem):
        return [
            pltpu.async_copy(y_hbm.at[idx_v.at[j * _GPC + q]],
                             buf.at[pl.ds(q * 128, 128)], sem)
            for q in range(_GPC)
        ]

    pending = fire(0, bufs[0], sems[0])
    for j in range(_PER_TILE):
        nxt = (fire(j + 1, bufs[(j + 1) % 2], sems[(j + 1) % 2])
               if j + 1 < _PER_TILE else [])
        for cp in pending:
            cp.wait()
        rows_v = bufs[j % 2]

        def group_body(g, inner):
            seg16 = seg_v[pl.ds(j * _CH + g * 16, 16)]
            base_r = g * 16
            for rr in range(16):
                off = seg16[rr] * _SEGW
                acc_max = maxs[rr % 2]
                for h in range(2):
                    v = rows_v[base_r + rr, pl.ds(h * 16, 16)]
                    sl = pl.ds(off + h * 16, 16)
                    plsc.addupdate(acc_sum.at[sl], v)
                    acc_max[sl] = jnp.maximum(acc_max[sl], v)
            return inner

        lax.fori_loop(0, _CH // 16, group_body, 0)
        pending = nxt

    def merge_body(i, carry):
        sl = pl.ds(i * 16, 16)
        acc_max_a[sl] = jnp.maximum(acc_max_a[sl], acc_max_b[sl])
        return carry

    lax.fori_loop(0, _ACC // 16, merge_body, 0)
    pltpu.sync_copy(acc_sum, sum_out.at[wid])
    pltpu.sync_copy(acc_max_a, max_out.at[wid])


def _run_segment_reduce(y_pad, idx2d, seg1d):
    mesh = plsc.VectorSubcoreMesh(core_axis_name="c", subcore_axis_name="s")
    f = pl.kernel(
        _seg_body,
        out_type=[jax.ShapeDtypeStruct((32, _ACC), jnp.float32),
                  jax.ShapeDtypeStruct((32, _ACC), jnp.float32)],
        mesh=mesh,
        scratch_types=[
            pltpu.VMEM((_ACC,), jnp.float32),
            pltpu.VMEM((_ACC,), jnp.float32),
            pltpu.VMEM((_ACC,), jnp.float32),
            pltpu.VMEM((_GPC * _PER_TILE, 128), jnp.int32),
            pltpu.VMEM((_CH * _PER_TILE,), jnp.int32),
            pltpu.VMEM((_CH, _SEGW), jnp.float32),
            pltpu.VMEM((_CH, _SEGW), jnp.float32),
            pltpu.SemaphoreType.DMA,
            pltpu.SemaphoreType.DMA,
        ],
        compiler_params=pltpu.CompilerParams(use_tc_tiling_on_sc=False),
    )
    return f(y_pad, idx2d, seg1d)


def _final_body(ps_ref, pm_ref, wo_ref, bo_ref, out_ref):
    sums = ps_ref[...].sum(axis=0)
    maxs = pm_ref[...].max(axis=0)
    counts = sums[:_C, 20:21]
    mean = sums[:_C, :20] / jnp.maximum(counts, 1.0)
    pooled = jnp.concatenate([mean, maxs[:_C, :20]], axis=1)
    out_ref[...] = jnp.maximum(
        jnp.dot(pooled, wo_ref[...], preferred_element_type=jnp.float32)
        + bo_ref[...], 0.0)


def _run_final(sum_p, max_p, w_out, b_out):
    return pl.pallas_call(
        _final_body,
        out_shape=jax.ShapeDtypeStruct((_C, _OUT), jnp.float32),
    )(sum_p.reshape(32, _C + _ND, _SEGW), max_p.reshape(32, _C + _ND, _SEGW),
      w_out, b_out.reshape(1, _OUT))


def kernel(x, dataset_x, community, multi_community_nodes,
           multi_community_index, W_dem, b_dem, W_pur, b_pur, W_feat, b_feat,
           W_out, b_out):
    y_pad = _run_mlp(x, dataset_x, W_dem, b_dem, W_pur, b_pur, W_feat, b_feat)

    pad = _TOT - (_N + _M)
    seg1d = jnp.concatenate([
        community.astype(jnp.int32),
        multi_community_index.astype(jnp.int32),
        _C + (jnp.arange(pad, dtype=jnp.int32) % _ND),
    ])
    idx2d = jnp.concatenate([
        jnp.arange(_N, dtype=jnp.int32),
        multi_community_nodes.astype(jnp.int32),
        jnp.zeros((pad,), jnp.int32),
    ]).reshape(_TOT // 128, 128)

    sum_p, max_p = _run_segment_reduce(y_pad, idx2d, seg1d)
    return _run_final(sum_p, max_p, W_out, b_out)


# speedup vs baseline: 1.2326x; 1.2326x over previous
import functools

import jax
import jax.numpy as jnp
from jax import lax
from jax.experimental import pallas as pl
from jax.experimental.pallas import tpu as pltpu
from jax.experimental.pallas import tpu_sc as plsc

_N = 100000
_M = 20000
_C = 1000
_OUT = 16
_SEGW = 32
_NSEG = 1024
_ND = _NSEG - _C
_CH = 256
_NCHUNK = 480
_TOT = _NCHUNK * _CH
_PER_TILE = _NCHUNK // 32
_GPC = _CH // 128
_RPT = _NSEG // 16


def _mlp_body(x_ref, dx_ref, wd_ref, bd_ref, wp_ref, bp_ref, wf_ref, bf_ref,
              out_ref):
    x = x_ref[...]
    dx = dx_ref[...]
    dem = jnp.maximum(
        jnp.dot(dx[:, :8], wd_ref[...], preferred_element_type=jnp.float32)
        + bd_ref[...], 0.0)
    pur = jnp.maximum(
        jnp.dot(dx[:, 8:], wp_ref[...], preferred_element_type=jnp.float32)
        + bp_ref[...], 0.0)
    y60 = jnp.concatenate([dem, pur, x], axis=1)
    y = jnp.maximum(
        jnp.dot(y60, wf_ref[...], preferred_element_type=jnp.float32)
        + bf_ref[...], 0.0)
    b = y.shape[0]
    yp = jnp.concatenate(
        [y, jnp.ones((b, 1), jnp.float32), jnp.zeros((b, 11), jnp.float32)],
        axis=1)
    out_ref[...] = yp


def _run_mlp(x, dataset_x, w_dem, b_dem, w_pur, b_pur, w_feat, b_feat):
    blk = 4000
    grid = _N // blk
    full = lambda shape: pl.BlockSpec(shape, lambda i: (0, 0))
    return pl.pallas_call(
        _mlp_body,
        grid=(grid,),
        in_specs=[
            pl.BlockSpec((blk, 20), lambda i: (i, 0)),
            pl.BlockSpec((blk, 20), lambda i: (i, 0)),
            full((8, 20)), full((1, 20)),
            full((12, 20)), full((1, 20)),
            full((60, 20)), full((1, 20)),
        ],
        out_specs=pl.BlockSpec((blk, _SEGW), lambda i: (i, 0)),
        out_shape=jax.ShapeDtypeStruct((_N, _SEGW), jnp.float32),
    )(x, dataset_x, w_dem, b_dem.reshape(1, 20), w_pur, b_pur.reshape(1, 20),
      w_feat, b_feat.reshape(1, 20))


def _seg_body(y_hbm, idx_hbm, seg_hbm, ididx_hbm, sum_out, max_out,
              acc_sum, acc_max, idx_v, seg_v, rows_a, rows_b, ididx_v,
              mtmp, mred, shsum, shmax, sem_a, sem_b):
    cid = lax.axis_index("c")
    sid = lax.axis_index("s")
    wid = cid * 16 + sid
    zero16 = jnp.zeros((16,), jnp.float32)

    idx_cp = pltpu.async_copy(idx_hbm.at[pl.ds(wid * _GPC * _PER_TILE,
                                               _GPC * _PER_TILE)], idx_v, sem_a)
    seg_cp = pltpu.async_copy(seg_hbm.at[pl.ds(wid * _CH * _PER_TILE,
                                               _CH * _PER_TILE)], seg_v, sem_b)
    pltpu.sync_copy(ididx_hbm, ididx_v)

    def init_body(i, carry):
        acc_sum[i, pl.ds(0, 16)] = zero16
        acc_sum[i, pl.ds(16, 16)] = zero16
        acc_max[i, pl.ds(0, 16)] = zero16
        acc_max[i, pl.ds(16, 16)] = zero16
        return carry

    lax.fori_loop(0, _NSEG, init_body, 0)
    idx_cp.wait()
    seg_cp.wait()

    bufs = (rows_a, rows_b)
    sems = (sem_a, sem_b)

    def fire(j, buf, sem):
        return [
            pltpu.async_copy(y_hbm.at[idx_v.at[j * _GPC + q]],
                             buf.at[pl.ds(q * 128, 128)], sem)
            for q in range(_GPC)
        ]

    pending = fire(0, bufs[0], sems[0])
    for j in range(_PER_TILE):
        nxt = (fire(j + 1, bufs[(j + 1) % 2], sems[(j + 1) % 2])
               if j + 1 < _PER_TILE else [])
        for cp in pending:
            cp.wait()
        rows_v = bufs[j % 2]

        def group_body(g, inner):
            seg16 = seg_v[pl.ds(j * _CH + g * 16, 16)]
            base_r = g * 16
            for rr in range(16):
                s = seg16[rr]
                for h in range(2):
                    v = rows_v[base_r + rr, pl.ds(h * 16, 16)]
                    sl = pl.ds(h * 16, 16)
                    plsc.addupdate(acc_sum.at[s, sl], v)
                    acc_max[s, sl] = jnp.maximum(acc_max[s, sl], v)
            return inner

        lax.fori_loop(0, _CH // 16, group_body, 0)
        pending = nxt

    @pl.when(sid == 0)
    def _():
        pltpu.sync_copy(acc_sum, shsum)
    pltpu.sync_copy(acc_max, shmax.at[sid])
    plsc.subcore_barrier()

    @pl.when(sid != 0)
    def _():
        pltpu.sync_copy(acc_sum, shsum.at[ididx_v], add=True)
    plsc.subcore_barrier()

    base = sid * _RPT
    pltpu.sync_copy(shmax.at[0, pl.ds(base, _RPT)], mred)

    def red_partial(p, carry):
        pltpu.sync_copy(shmax.at[p, pl.ds(base, _RPT)], mtmp)

        def red_row(i, inner):
            for h in range(2):
                sl = pl.ds(h * 16, 16)
                mred[i, sl] = jnp.maximum(mred[i, sl], mtmp[i, sl])
            return inner

        lax.fori_loop(0, _RPT, red_row, 0)
        return carry

    lax.fori_loop(1, 16, red_partial, 0)
    pltpu.sync_copy(mred, max_out.at[cid, pl.ds(base, _RPT)])
    pltpu.sync_copy(shsum.at[pl.ds(base, _RPT)],
                    sum_out.at[cid, pl.ds(base, _RPT)])


def _run_segment_reduce(y_pad, idx2d, seg1d, ididx):
    mesh = plsc.VectorSubcoreMesh(core_axis_name="c", subcore_axis_name="s")
    f = pl.kernel(
        _seg_body,
        out_type=[jax.ShapeDtypeStruct((2, _NSEG, _SEGW), jnp.float32),
                  jax.ShapeDtypeStruct((2, _NSEG, _SEGW), jnp.float32)],
        mesh=mesh,
        scratch_types=[
            pltpu.VMEM((_NSEG, _SEGW), jnp.float32),
            pltpu.VMEM((_NSEG, _SEGW), jnp.float32),
            pltpu.VMEM((_GPC * _PER_TILE, 128), jnp.int32),
            pltpu.VMEM((_CH * _PER_TILE,), jnp.int32),
            pltpu.VMEM((_CH, _SEGW), jnp.float32),
            pltpu.VMEM((_CH, _SEGW), jnp.float32),
            pltpu.VMEM((_NSEG,), jnp.int32),
            pltpu.VMEM((_RPT, _SEGW), jnp.float32),
            pltpu.VMEM((_RPT, _SEGW), jnp.float32),
            pltpu.VMEM_SHARED((_NSEG, _SEGW), jnp.float32),
            pltpu.VMEM_SHARED((16, _NSEG, _SEGW), jnp.float32),
            pltpu.SemaphoreType.DMA,
            pltpu.SemaphoreType.DMA,
        ],
        compiler_params=pltpu.CompilerParams(use_tc_tiling_on_sc=False),
    )
    return f(y_pad, idx2d, seg1d, ididx)


def _final_body(ps_ref, pm_ref, wo_ref, bo_ref, out_ref):
    sums = ps_ref[...].sum(axis=0)
    maxs = pm_ref[...].max(axis=0)
    counts = sums[:_C, 20:21]
    mean = sums[:_C, :20] / jnp.maximum(counts, 1.0)
    pooled = jnp.concatenate([mean, maxs[:_C, :20]], axis=1)
    out_ref[...] = jnp.maximum(
        jnp.dot(pooled, wo_ref[...], preferred_element_type=jnp.float32)
        + bo_ref[...], 0.0)


def _run_final(sum_p, max_p, w_out, b_out):
    return pl.pallas_call(
        _final_body,
        out_shape=jax.ShapeDtypeStruct((_C, _OUT), jnp.float32),
    )(sum_p, max_p, w_out, b_out.reshape(1, _OUT))


def kernel(x, dataset_x, community, multi_community_nodes,
           multi_community_index, W_dem, b_dem, W_pur, b_pur, W_feat, b_feat,
           W_out, b_out):
    y_pad = _run_mlp(x, dataset_x, W_dem, b_dem, W_pur, b_pur, W_feat, b_feat)

    pad = _TOT - (_N + _M)
    seg1d = jnp.concatenate([
        community.astype(jnp.int32),
        multi_community_index.astype(jnp.int32),
        _C + (jnp.arange(pad, dtype=jnp.int32) % _ND),
    ])
    idx2d = jnp.concatenate([
        jnp.arange(_N, dtype=jnp.int32),
        multi_community_nodes.astype(jnp.int32),
        jnp.zeros((pad,), jnp.int32),
    ]).reshape(_TOT // 128, 128)
    ididx = jnp.arange(_NSEG, dtype=jnp.int32)

    sum_p, max_p = _run_segment_reduce(y_pad, idx2d, seg1d, ididx)
    return _run_final(sum_p, max_p, W_out, b_out)


# speedup vs baseline: 1.2337x; 1.0009x over previous
import jax
import jax.numpy as jnp
from jax import lax
from jax.experimental import pallas as pl
from jax.experimental.pallas import tpu as pltpu
from jax.experimental.pallas import tpu_sc as plsc

_N = 100000
_M = 20000
_C = 1000
_OUT = 16
_SEGW = 32
_NSEG = 1024
_ND = _NSEG - _C
_CH = 256
_NCHUNK = 480
_TOT = _NCHUNK * _CH
_PER_TILE = _NCHUNK // 32
_GPC = _CH // 128
_RPT = _NSEG // 16


def _mlp_body(x_ref, dx_ref, wd_ref, bd_ref, wp_ref, bp_ref, wf_ref, bf_ref,
              out_ref):
    x = x_ref[...]
    dx = dx_ref[...]
    dem = jnp.maximum(
        jnp.dot(dx[:, :8], wd_ref[...], preferred_element_type=jnp.float32)
        + bd_ref[...], 0.0)
    pur = jnp.maximum(
        jnp.dot(dx[:, 8:], wp_ref[...], preferred_element_type=jnp.float32)
        + bp_ref[...], 0.0)
    y60 = jnp.concatenate([dem, pur, x], axis=1)
    y = jnp.maximum(
        jnp.dot(y60, wf_ref[...], preferred_element_type=jnp.float32)
        + bf_ref[...], 0.0)
    b = y.shape[0]
    yp = jnp.concatenate(
        [y, jnp.ones((b, 1), jnp.float32), jnp.zeros((b, 11), jnp.float32)],
        axis=1)
    out_ref[...] = yp


def _run_mlp(x, dataset_x, w_dem, b_dem, w_pur, b_pur, w_feat, b_feat):
    blk = 4000
    grid = _N // blk
    full = lambda shape: pl.BlockSpec(shape, lambda i: (0, 0))
    return pl.pallas_call(
        _mlp_body,
        grid=(grid,),
        in_specs=[
            pl.BlockSpec((blk, 20), lambda i: (i, 0)),
            pl.BlockSpec((blk, 20), lambda i: (i, 0)),
            full((8, 20)), full((1, 20)),
            full((12, 20)), full((1, 20)),
            full((60, 20)), full((1, 20)),
        ],
        out_specs=pl.BlockSpec((blk, _SEGW), lambda i: (i, 0)),
        out_shape=jax.ShapeDtypeStruct((_N, _SEGW), jnp.float32),
    )(x, dataset_x, w_dem, b_dem.reshape(1, 20), w_pur, b_pur.reshape(1, 20),
      w_feat, b_feat.reshape(1, 20))


def _seg_body(y_hbm, idx_hbm, seg_hbm, ididx_hbm, sum_out, max_out,
              acc_sum, acc_max, idx_v, seg_v, rows_a, rows_b, ididx_v,
              mtmp, mred, shsum, shmax, sem_a, sem_b):
    cid = lax.axis_index("c")
    sid = lax.axis_index("s")
    wid = cid * 16 + sid
    zero16 = jnp.zeros((16,), jnp.float32)

    idx_cp = pltpu.async_copy(idx_hbm.at[pl.ds(wid * _GPC * _PER_TILE,
                                               _GPC * _PER_TILE)], idx_v, sem_a)
    seg_cp = pltpu.async_copy(seg_hbm.at[pl.ds(wid * _CH * _PER_TILE,
                                               _CH * _PER_TILE)], seg_v, sem_b)
    pltpu.sync_copy(ididx_hbm, ididx_v)

    def init_body(i, carry):
        acc_sum[i, pl.ds(0, 16)] = zero16
        acc_sum[i, pl.ds(16, 16)] = zero16
        acc_max[i, pl.ds(0, 16)] = zero16
        acc_max[i, pl.ds(16, 16)] = zero16
        return carry

    lax.fori_loop(0, _NSEG, init_body, 0)
    idx_cp.wait()
    seg_cp.wait()

    bufs = (rows_a, rows_b)
    sems = (sem_a, sem_b)

    def fire(j, buf, sem):
        return [
            pltpu.async_copy(y_hbm.at[idx_v.at[j * _GPC + q]],
                             buf.at[pl.ds(q * 128, 128)], sem)
            for q in range(_GPC)
        ]

    pending = fire(0, bufs[0], sems[0])
    for j in range(_PER_TILE):
        nxt = (fire(j + 1, bufs[(j + 1) % 2], sems[(j + 1) % 2])
               if j + 1 < _PER_TILE else [])
        for cp in pending:
            cp.wait()
        rows_v = bufs[j % 2]

        def group_body(g, inner):
            seg16 = seg_v[pl.ds(j * _CH + g * 16, 16)]
            base_r = g * 16
            for rr in range(16):
                s = seg16[rr]
                for h in range(2):
                    v = rows_v[base_r + rr, pl.ds(h * 16, 16)]
                    sl = pl.ds(h * 16, 16)
                    plsc.addupdate(acc_sum.at[s, sl], v)
                    acc_max[s, sl] = jnp.maximum(acc_max[s, sl], v)
            return inner

        lax.fori_loop(0, _CH // 16, group_body, 0)
        pending = nxt

    @pl.when(sid == 0)
    def _():
        pltpu.sync_copy(acc_sum, shsum)
    pltpu.sync_copy(acc_max, shmax.at[sid])
    plsc.subcore_barrier()

    @pl.when(sid != 0)
    def _():
        pltpu.sync_copy(acc_sum, shsum.at[ididx_v], add=True)
    plsc.subcore_barrier()

    base = sid * _RPT
    pltpu.sync_copy(shmax.at[0, pl.ds(base, _RPT)], mred)

    def red_partial(p, carry):
        pltpu.sync_copy(shmax.at[p, pl.ds(base, _RPT)], mtmp)

        def red_row(i, inner):
            for h in range(2):
                sl = pl.ds(h * 16, 16)
                mred[i, sl] = jnp.maximum(mred[i, sl], mtmp[i, sl])
            return inner

        lax.fori_loop(0, _RPT, red_row, 0)
        return carry

    lax.fori_loop(1, 16, red_partial, 0)
    pltpu.sync_copy(mred, max_out.at[cid, pl.ds(base, _RPT)])
    pltpu.sync_copy(shsum.at[pl.ds(base, _RPT)],
                    sum_out.at[cid, pl.ds(base, _RPT)])


def _run_segment_reduce(y_pad, idx2d, seg1d, ididx):
    mesh = plsc.VectorSubcoreMesh(core_axis_name="c", subcore_axis_name="s")
    f = pl.kernel(
        _seg_body,
        out_type=[jax.ShapeDtypeStruct((2, _NSEG, _SEGW), jnp.float32),
                  jax.ShapeDtypeStruct((2, _NSEG, _SEGW), jnp.float32)],
        mesh=mesh,
        scratch_types=[
            pltpu.VMEM((_NSEG, _SEGW), jnp.float32),
            pltpu.VMEM((_NSEG, _SEGW), jnp.float32),
            pltpu.VMEM((_GPC * _PER_TILE, 128), jnp.int32),
            pltpu.VMEM((_CH * _PER_TILE,), jnp.int32),
            pltpu.VMEM((_CH, _SEGW), jnp.float32),
            pltpu.VMEM((_CH, _SEGW), jnp.float32),
            pltpu.VMEM((_NSEG,), jnp.int32),
            pltpu.VMEM((_RPT, _SEGW), jnp.float32),
            pltpu.VMEM((_RPT, _SEGW), jnp.float32),
            pltpu.VMEM_SHARED((_NSEG, _SEGW), jnp.float32),
            pltpu.VMEM_SHARED((16, _NSEG, _SEGW), jnp.float32),
            pltpu.SemaphoreType.DMA,
            pltpu.SemaphoreType.DMA,
        ],
        compiler_params=pltpu.CompilerParams(use_tc_tiling_on_sc=False),
    )
    return f(y_pad, idx2d, seg1d, ididx)


def _final_body(ps_ref, pm_ref, wo_ref, bo_ref, out_ref):
    sums = ps_ref[...].sum(axis=0)
    maxs = pm_ref[...].max(axis=0)
    counts = sums[:_C, 20:21]
    mean = sums[:_C, :20] / jnp.maximum(counts, 1.0)
    pooled = jnp.concatenate([mean, maxs[:_C, :20]], axis=1)
    out_ref[...] = jnp.maximum(
        jnp.dot(pooled, wo_ref[...], preferred_element_type=jnp.float32)
        + bo_ref[...], 0.0)


def _run_final(sum_p, max_p, w_out, b_out):
    return pl.pallas_call(
        _final_body,
        out_shape=jax.ShapeDtypeStruct((_C, _OUT), jnp.float32),
    )(sum_p, max_p, w_out, b_out.reshape(1, _OUT))


def kernel(x, dataset_x, community, multi_community_nodes,
           multi_community_index, W_dem, b_dem, W_pur, b_pur, W_feat, b_feat,
           W_out, b_out):
    y_pad = _run_mlp(x, dataset_x, W_dem, b_dem, W_pur, b_pur, W_feat, b_feat)

    pad = _TOT - (_N + _M)
    seg1d = jnp.concatenate([
        community.astype(jnp.int32),
        multi_community_index.astype(jnp.int32),
        _C + (jnp.arange(pad, dtype=jnp.int32) % _ND),
    ])
    idx2d = jnp.concatenate([
        jnp.arange(_N, dtype=jnp.int32),
        multi_community_nodes.astype(jnp.int32),
        jnp.zeros((pad,), jnp.int32),
    ]).reshape(_TOT // 128, 128)
    ididx = jnp.arange(_NSEG, dtype=jnp.int32)

    sum_p, max_p = _run_segment_reduce(y_pad, idx2d, seg1d, ididx)
    return _run_final(sum_p, max_p, W_out, b_out)
